# baseline (speedup 1.0000x reference)
import jax
import jax.numpy as jnp
from jax import lax
from jax.experimental import pallas as pl
from jax.experimental.pallas import tpu as pltpu
from jax.experimental.pallas import tpu_sc as plsc

_B = 16384
_D = 64
_NC = 2
_NS = 16
_NW = _NC * _NS
_PER_W = _B // _NW
_HP = _PER_W // 2
_UNROLL = 16


def _body(s_hbm, r_hbm, o_hbm, e_hbm, rel_hbm, out_hbm,
          idx_sh, s_rows, r_rows, o_rows, out_v,
          s_sm, r_sm, o_sm, sem):
    cid = lax.axis_index("c")
    sid = lax.axis_index("s")
    wid = sid * _NC + cid
    base = wid * _PER_W

    for hbm, sm in ((s_hbm, s_sm), (r_hbm, r_sm), (o_hbm, o_sm)):
        pltpu.sync_copy(hbm.at[pl.ds(base, _PER_W)], idx_sh.at[sid])
        pltpu.sync_copy(idx_sh.at[sid], sm)

    lanes = lax.iota(jnp.int32, 16)

    for p in range(2):
        poff = p * _HP

        def fire(i, _, poff=poff):
            for k in range(_UNROLL):
                row = i * _UNROLL + k
                dst = pl.ds(row, 1)
                pltpu.async_copy(e_hbm.at[pl.ds(s_sm[poff + row], 1)],
                                 s_rows.at[dst], sem)
                pltpu.async_copy(rel_hbm.at[pl.ds(r_sm[poff + row], 1)],
                                 r_rows.at[dst], sem)
                pltpu.async_copy(e_hbm.at[pl.ds(o_sm[poff + row], 1)],
                                 o_rows.at[dst], sem)
            return 0

        lax.fori_loop(0, _HP // _UNROLL, fire, 0)
        pltpu.make_async_copy(e_hbm.at[pl.ds(0, _HP)], s_rows, sem).wait()
        pltpu.make_async_copy(e_hbm.at[pl.ds(0, _HP)], r_rows, sem).wait()
        pltpu.make_async_copy(e_hbm.at[pl.ds(0, _HP)], o_rows, sem).wait()

        for g in range(_HP // 16):
            rows = g * 16 + lanes

            def j_body(i, acc, rows=rows):
                for k in range(4):
                    col = jnp.full((16,), 0, jnp.int32) + (i * 4 + k)
                    sv = plsc.load_gather(s_rows, [rows, col])
                    rv = plsc.load_gather(r_rows, [rows, col])
                    ov = plsc.load_gather(o_rows, [rows, col])
                    acc = acc + jnp.abs(sv + rv - ov)
                return acc

            acc = lax.fori_loop(0, _D // 4, j_body, jnp.zeros((16,), jnp.float32))
            out_v[pl.ds(poff + g * 16, 16)] = -acc

    pltpu.sync_copy(out_v, out_hbm.at[pl.ds(base, _PER_W)])


@jax.jit
def _transe_sc(s, r, o, e, rel):
    mesh = plsc.VectorSubcoreMesh(core_axis_name="c", subcore_axis_name="s")
    return pl.kernel(
        _body,
        mesh=mesh,
        compiler_params=pltpu.CompilerParams(
            needs_layout_passes=False, use_tc_tiling_on_sc=True),
        out_type=jax.ShapeDtypeStruct((_B,), jnp.float32),
        scratch_types=[
            pltpu.VMEM_SHARED((_NS, _PER_W), jnp.int32),
            pltpu.VMEM((_HP, _D), jnp.float32),
            pltpu.VMEM((_HP, _D), jnp.float32),
            pltpu.VMEM((_HP, _D), jnp.float32),
            pltpu.VMEM((_PER_W,), jnp.float32),
            pltpu.SMEM((_PER_W,), jnp.int32),
            pltpu.SMEM((_PER_W,), jnp.int32),
            pltpu.SMEM((_PER_W,), jnp.int32),
            pltpu.SemaphoreType.DMA,
        ],
    )(s, r, o, e, rel)


def kernel(s, r, o, E_center, R_center):
    e_b = lax.optimization_barrier(E_center)
    return _transe_sc(s, r, o, e_b, R_center)


# speedup vs baseline: 1.0010x; 1.0010x over previous
import jax
import jax.numpy as jnp
from jax import lax
from jax.experimental import pallas as pl
from jax.experimental.pallas import tpu as pltpu
from jax.experimental.pallas import tpu_sc as plsc

_B = 16384
_D = 64
_NC = 2
_NS = 16
_NW = _NC * _NS
_PER_W = _B // _NW
_HP = _PER_W // 2
_UNROLL = 16


def _body(s_hbm, r_hbm, o_hbm, e_hbm, rel_hbm, out_hbm,
          idx_sh, s_rows, r_rows, o_rows, out_v,
          s_sm, r_sm, o_sm, sem):
    cid = lax.axis_index("c")
    sid = lax.axis_index("s")
    wid = sid * _NC + cid
    base = wid * _PER_W

    for hbm, sm in ((s_hbm, s_sm), (r_hbm, r_sm), (o_hbm, o_sm)):
        pltpu.sync_copy(hbm.at[pl.ds(base, _PER_W)], idx_sh.at[sid])
        pltpu.sync_copy(idx_sh.at[sid], sm)

    lanes = lax.iota(jnp.int32, 16)

    for p in range(2):
        poff = p * _HP

        def fire(i, _, poff=poff):
            for k in range(_UNROLL):
                row = i * _UNROLL + k
                dst = pl.ds(row, 1)
                pltpu.async_copy(e_hbm.at[pl.ds(s_sm[poff + row], 1)],
                                 s_rows.at[dst], sem)
                pltpu.async_copy(rel_hbm.at[pl.ds(r_sm[poff + row], 1)],
                                 r_rows.at[dst], sem)
                pltpu.async_copy(e_hbm.at[pl.ds(o_sm[poff + row], 1)],
                                 o_rows.at[dst], sem)
            return 0

        lax.fori_loop(0, _HP // _UNROLL, fire, 0)
        pltpu.make_async_copy(e_hbm.at[pl.ds(0, _HP)], s_rows, sem).wait()
        pltpu.make_async_copy(e_hbm.at[pl.ds(0, _HP)], r_rows, sem).wait()
        pltpu.make_async_copy(e_hbm.at[pl.ds(0, _HP)], o_rows, sem).wait()

        for g in range(_HP // 16):
            rows = g * 16 + lanes

            def j_body(i, acc, rows=rows):
                for k in range(4):
                    col = jnp.full((16,), 0, jnp.int32) + (i * 4 + k)
                    sv = plsc.load_gather(s_rows, [rows, col])
                    rv = plsc.load_gather(r_rows, [rows, col])
                    ov = plsc.load_gather(o_rows, [rows, col])
                    acc = acc + jnp.abs(sv + rv - ov)
                return acc

            acc = lax.fori_loop(0, _D // 4, j_body, jnp.zeros((16,), jnp.float32))
            out_v[pl.ds(poff + g * 16, 16)] = -acc

    pltpu.sync_copy(out_v, out_hbm.at[pl.ds(base, _PER_W)])


@jax.jit
def _transe_sc(s, r, o, e, rel):
    mesh = plsc.VectorSubcoreMesh(core_axis_name="c", subcore_axis_name="s")
    return pl.kernel(
        _body,
        mesh=mesh,
        compiler_params=pltpu.CompilerParams(
            needs_layout_passes=False, use_tc_tiling_on_sc=True),
        out_type=jax.ShapeDtypeStruct((_B,), jnp.float32),
        scratch_types=[
            pltpu.VMEM_SHARED((_NS, _PER_W), jnp.int32),
            pltpu.VMEM((_HP, _D), jnp.float32),
            pltpu.VMEM((_HP, _D), jnp.float32),
            pltpu.VMEM((_HP, _D), jnp.float32),
            pltpu.VMEM((_PER_W,), jnp.float32),
            pltpu.SMEM((_PER_W,), jnp.int32),
            pltpu.SMEM((_PER_W,), jnp.int32),
            pltpu.SMEM((_PER_W,), jnp.int32),
            pltpu.SemaphoreType.DMA,
        ],
    )(s, r, o, e, rel)


def kernel(s, r, o, E_center, R_center):
    return _transe_sc(s, r, o, E_center, R_center)
